# Initial kernel scaffold; baseline (speedup 1.0000x reference)
import jax
import jax.numpy as jnp
from jax.experimental import pallas as pl


def kernel(input):
    raise NotImplementedError("write your pallas kernel here")


# speedup vs baseline: 3.4158x; 3.4158x over previous
import functools

import jax
import jax.numpy as jnp
from jax import lax
from jax.experimental import pallas as pl
from jax.experimental.pallas import tpu as pltpu
from jax.experimental.pallas import tpu_sc as plsc

B = 128
N = 8192
L = 16
NVEC = N // L


def _splat(x):
    return jnp.full((L,), x, jnp.float32)


def _row_fixpoint(xv, r):
    def sum_body(i, acc):
        v = xv[r, pl.ds(i * L, L)]
        return acc + v

    acc0 = lax.fori_loop(0, NVEC, sum_body, jnp.zeros((L,), jnp.float32))
    total = _splat(jnp.sum(acc0))
    t0 = (total - 1.0) / _splat(jnp.float32(N))

    def cond(carry):
        _, k, prev_k = carry
        return jnp.any(k != prev_k)

    def body(carry):
        t, k, _ = carry

        def pass_body(i, accs):
            sacc, cacc = accs
            v = xv[r, pl.ds(i * L, L)]
            m = v > t
            sacc = sacc + jnp.where(m, v, jnp.float32(0.0))
            cacc = cacc + jnp.where(m, jnp.float32(1.0), jnp.float32(0.0))
            return sacc, cacc

        zeros = jnp.zeros((L,), jnp.float32)
        sacc, cacc = lax.fori_loop(0, NVEC, pass_body, (zeros, zeros))
        s_new = _splat(jnp.sum(sacc))
        k_new = _splat(jnp.sum(cacc))
        t_new = (s_new - 1.0) / k_new
        return t_new, k_new, k

    t, k, _ = lax.while_loop(
        cond, body, (t0, _splat(jnp.float32(N)), _splat(jnp.float32(-1.0)))
    )
    return t - 1.0 / k


def _make_sc_kernel():
    info = plsc.get_sparse_core_info()
    nw = info.num_cores * info.num_subcores
    rows_per_w = B // nw
    mesh = plsc.VectorSubcoreMesh(core_axis_name="c", subcore_axis_name="s")

    @functools.partial(
        pl.kernel,
        mesh=mesh,
        out_type=jax.ShapeDtypeStruct((B, N), jnp.float32),
        scratch_types=[pltpu.VMEM((rows_per_w, N), jnp.float32)],
        compiler_params=pltpu.CompilerParams(needs_layout_passes=False),
    )
    def sparsemax_sc(x_hbm, out_hbm, xv):
        wid = lax.axis_index("s") * info.num_cores + lax.axis_index("c")
        base = wid * rows_per_w
        pltpu.sync_copy(x_hbm.at[pl.ds(base, rows_per_w)], xv)
        for r in range(rows_per_w):
            tau = _row_fixpoint(xv, r)

            def out_body(i, _, r=r, tau=tau):
                v = xv[r, pl.ds(i * L, L)]
                xv[r, pl.ds(i * L, L)] = jnp.maximum(v - tau, jnp.float32(0.0))
                return 0

            lax.fori_loop(0, NVEC, out_body, 0)
        pltpu.sync_copy(xv, out_hbm.at[pl.ds(base, rows_per_w)])

    return sparsemax_sc


_sparsemax = _make_sc_kernel()


def kernel(input):
    return _sparsemax(input)


# speedup vs baseline: 6.8834x; 2.0151x over previous
import functools

import jax
import jax.numpy as jnp
from jax import lax
from jax.experimental import pallas as pl
from jax.experimental.pallas import tpu as pltpu
from jax.experimental.pallas import tpu_sc as plsc

B = 128
N = 8192
L = 16
NVEC = N // L


def _splat(x):
    return jnp.full((L,), x, jnp.float32)


def _row_fixpoint(xv, r):
    def sum_body(i, accs):
        base = i * (4 * L)
        return tuple(
            acc + xv[r, pl.ds(base + j * L, L)] for j, acc in enumerate(accs)
        )

    zero = jnp.zeros((L,), jnp.float32)
    accs0 = lax.fori_loop(0, NVEC // 4, sum_body, (zero, zero, zero, zero))
    acc0 = (accs0[0] + accs0[1]) + (accs0[2] + accs0[3])
    total = _splat(jnp.sum(acc0))
    t0 = (total - 1.0) / _splat(jnp.float32(N))

    def cond(carry):
        _, k, prev_k = carry
        return jnp.any(k != prev_k)

    def body(carry):
        t, k, _ = carry

        def pass_body(i, accs):
            base = i * (4 * L)
            out = []
            for j in range(4):
                sacc, cacc = accs[2 * j], accs[2 * j + 1]
                v = xv[r, pl.ds(base + j * L, L)]
                m = v > t
                out.append(sacc + jnp.where(m, v, jnp.float32(0.0)))
                out.append(cacc + jnp.where(m, jnp.float32(1.0), jnp.float32(0.0)))
            return tuple(out)

        zero = jnp.zeros((L,), jnp.float32)
        accs = lax.fori_loop(0, NVEC // 4, pass_body, (zero,) * 8)
        sacc = (accs[0] + accs[2]) + (accs[4] + accs[6])
        cacc = (accs[1] + accs[3]) + (accs[5] + accs[7])
        s_new = _splat(jnp.sum(sacc))
        k_new = _splat(jnp.sum(cacc))
        t_new = (s_new - 1.0) / k_new
        return t_new, k_new, k

    t, k, _ = lax.while_loop(
        cond, body, (t0, _splat(jnp.float32(N)), _splat(jnp.float32(-1.0)))
    )
    return t - 1.0 / k


def _make_sc_kernel():
    info = plsc.get_sparse_core_info()
    nw = info.num_cores * info.num_subcores
    rows_per_w = B // nw
    mesh = plsc.VectorSubcoreMesh(core_axis_name="c", subcore_axis_name="s")

    @functools.partial(
        pl.kernel,
        mesh=mesh,
        out_type=jax.ShapeDtypeStruct((B, N), jnp.float32),
        scratch_types=[pltpu.VMEM((rows_per_w, N), jnp.float32)],
        compiler_params=pltpu.CompilerParams(needs_layout_passes=False),
    )
    def sparsemax_sc(x_hbm, out_hbm, xv):
        wid = lax.axis_index("s") * info.num_cores + lax.axis_index("c")
        base = wid * rows_per_w
        pltpu.sync_copy(x_hbm.at[pl.ds(base, rows_per_w)], xv)
        for r in range(rows_per_w):
            tau = _row_fixpoint(xv, r)

            def out_body(i, _, r=r, tau=tau):
                base = i * (4 * L)
                for j in range(4):
                    v = xv[r, pl.ds(base + j * L, L)]
                    xv[r, pl.ds(base + j * L, L)] = jnp.maximum(
                        v - tau, jnp.float32(0.0)
                    )
                return 0

            lax.fori_loop(0, NVEC // 4, out_body, 0)
        pltpu.sync_copy(xv, out_hbm.at[pl.ds(base, rows_per_w)])

    return sparsemax_sc


_sparsemax = _make_sc_kernel()


def kernel(input):
    return _sparsemax(input)
